# baseline (speedup 1.0000x reference)
import functools

import jax
import jax.numpy as jnp
from jax import lax
from jax.experimental import pallas as pl
from jax.experimental.pallas import tpu as pltpu
from jax.experimental.pallas import tpu_sc as plsc

N_PAD = 64
PAIR = 2 * N_PAD
B_CHUNK = 32

_DN_T = (((1,), (1,)), ((), ()))


def _build_adj(ei, zeros, n_real):
    epad = ei.shape[1]
    nvec = ((n_real + 15) // 16)
    mesh = plsc.VectorSubcoreMesh(core_axis_name="c", subcore_axis_name="s")

    @functools.partial(
        pl.kernel,
        mesh=mesh,
        out_type=jax.ShapeDtypeStruct((N_PAD, N_PAD), jnp.float32),
        scratch_types=[
            pltpu.VMEM((N_PAD, N_PAD), jnp.float32),
            pltpu.VMEM((epad,), jnp.int32),
            pltpu.VMEM((epad,), jnp.int32),
        ],
        compiler_params=pltpu.CompilerParams(needs_layout_passes=False),
    )
    def k(ei_hbm, zeros_hbm, out_hbm, mbuf, srcv, dstv):
        cid = lax.axis_index("c")
        sid = lax.axis_index("s")

        @pl.when(jnp.logical_and(cid == 0, sid == 0))
        def _():
            pltpu.sync_copy(zeros_hbm, mbuf)
            pltpu.sync_copy(ei_hbm.at[0], srcv)
            pltpu.sync_copy(ei_hbm.at[1], dstv)
            lane = lax.iota(jnp.int32, 16)
            ones = jnp.ones((16,), jnp.float32)
            for c in range(nvec):
                s = srcv[pl.ds(c * 16, 16)]
                d = dstv[pl.ds(c * 16, 16)]
                for j in range(16):
                    plsc.addupdate_scatter(mbuf, [d, s], ones, mask=lane == j)
            pltpu.sync_copy(mbuf, out_hbm)

    return k(ei, zeros)


def _tc_body(data_ref, load_ref, m_ref, e1_ref, e2_ref, e3_ref,
             w1_ref, b1_ref, w2_ref, b2_ref, w3_ref, b3_ref,
             wl_ref, bl_ref, wo_ref, bo_ref, out_ref,
             a1s, a2s, a3s, s_scr, ys, ha, hb, feats):
    f32 = jnp.float32
    _DN_N = (((1,), (0,)), ((), ()))
    i = pl.program_id(0)
    rows = B_CHUNK * N_PAD

    @pl.when(i == 0)
    def _build_scratch():
        m = m_ref[...]
        r = lax.broadcasted_iota(jnp.int32, (N_PAD, N_PAD), 0)
        c = lax.broadcasted_iota(jnp.int32, (N_PAD, N_PAD), 1)
        dmask = jnp.logical_and(r == c, r < 53).astype(f32)
        for e_ref, a_scr in ((e1_ref, a1s), (e2_ref, a2s), (e3_ref, a3s)):
            a_scr[...] = m + (1.0 + e_ref[0, 0]) * dmask
        sr = lax.broadcasted_iota(jnp.int32, (B_CHUNK, rows), 0)
        sc = lax.broadcasted_iota(jnp.int32, (B_CHUNK, rows), 1)
        s_scr[...] = jnp.logical_and(sc // N_PAD == sr,
                                     sc % N_PAD < 53).astype(f32)

    for b in range(B_CHUNK):
        ys[pl.ds(b * N_PAD, 53)] = lax.dot_general(
            data_ref[b], w1_ref[...], _DN_T, preferred_element_type=f32)

    a1c = a1s[:, :53]
    b1 = b1_ref[...]
    for b in range(B_CHUNK):
        ha[pl.ds(b * N_PAD, N_PAD)] = jnp.maximum(
            lax.dot_general(a1c, ys[pl.ds(b * N_PAD, 53)], _DN_N,
                            preferred_element_type=f32) + b1, 0.0)
    feats[:, pl.ds(0, 256)] = lax.dot_general(
        s_scr[...], ha[...], _DN_N, preferred_element_type=f32)

    h_from, h_to = ha, hb
    for li, (w_ref, bref, a_scr) in enumerate(
            ((w2_ref, b2_ref, a2s), (w3_ref, b3_ref, a3s))):
        ys[...] = lax.dot_general(h_from[...], w_ref[...], _DN_T,
                                  preferred_element_type=f32)
        a = a_scr[...]
        bias = bref[...]
        for b in range(B_CHUNK):
            h_to[pl.ds(b * N_PAD, N_PAD)] = jnp.maximum(
                lax.dot_general(a, ys[pl.ds(b * N_PAD, N_PAD)], _DN_N,
                                preferred_element_type=f32) + bias, 0.0)
        feats[:, pl.ds(256 * (li + 1), 256)] = lax.dot_general(
            s_scr[...], h_to[...], _DN_N, preferred_element_type=f32)
        h_from, h_to = h_to, h_from

    lv = lax.dot_general(load_ref[...], wl_ref[...], _DN_T,
                         preferred_element_type=f32) + bl_ref[...]
    feats[:, pl.ds(768, 128)] = jnp.where(lv >= 0, lv, 0.01 * lv)

    out_ref[...] = (
        lax.dot_general(feats[...], wo_ref[...], _DN_T,
                        preferred_element_type=f32) + bo_ref[...])


def _tc_specs(bs):
    nsteps = bs // B_CHUNK
    fixed = lambda *_: tuple([0, 0])
    in_specs = [
        pl.BlockSpec((B_CHUNK, 53, 400), lambda i: (i, 0, 0)),
        pl.BlockSpec((B_CHUNK, 26), lambda i: (i, 0)),
        pl.BlockSpec((N_PAD, N_PAD), fixed),
        pl.BlockSpec((1, 1), fixed),
        pl.BlockSpec((1, 1), fixed),
        pl.BlockSpec((1, 1), fixed),
        pl.BlockSpec((256, 400), fixed),
        pl.BlockSpec((1, 256), fixed),
        pl.BlockSpec((256, 256), fixed),
        pl.BlockSpec((1, 256), fixed),
        pl.BlockSpec((256, 256), fixed),
        pl.BlockSpec((1, 256), fixed),
        pl.BlockSpec((128, 26), fixed),
        pl.BlockSpec((1, 128), fixed),
        pl.BlockSpec((2, 896), fixed),
        pl.BlockSpec((1, 2), fixed),
    ]
    out_spec = pl.BlockSpec((B_CHUNK, 2), lambda i: (i, 0))
    rows = B_CHUNK * N_PAD
    scratch = [pltpu.VMEM((N_PAD, N_PAD), jnp.float32) for _ in range(3)]
    scratch += [
        pltpu.VMEM((B_CHUNK, rows), jnp.float32),
        pltpu.VMEM((rows, 256), jnp.float32),
        pltpu.VMEM((rows, 256), jnp.float32),
        pltpu.VMEM((rows, 256), jnp.float32),
        pltpu.VMEM((B_CHUNK, 896), jnp.float32),
    ]
    return nsteps, in_specs, out_spec, scratch


def kernel(data, loading, edge_index, W1, b1, eps1, W2, b2, eps2,
           W3, b3, eps3, Wl, bl, Wo, bo):
    f32 = jnp.float32
    bs = data.shape[0]

    zeros = jnp.zeros((N_PAD, N_PAD), f32)
    n_real = edge_index.shape[1]
    epad = ((n_real + 127) // 128) * 128
    ei = jnp.pad(edge_index.astype(jnp.int32), ((0, 0), (0, epad - n_real)),
                 constant_values=N_PAD - 1)
    m = _build_adj(ei, zeros, n_real)

    nsteps, in_specs, out_spec, scratch = _tc_specs(bs)
    return pl.pallas_call(
        _tc_body,
        grid=(nsteps,),
        in_specs=in_specs,
        out_specs=out_spec,
        out_shape=jax.ShapeDtypeStruct((bs, 2), f32),
        scratch_shapes=scratch,
    )(data, loading, m, eps1.reshape(1, 1), eps2.reshape(1, 1),
      eps3.reshape(1, 1), W1, b1.reshape(1, -1), W2, b2.reshape(1, -1),
      W3, b3.reshape(1, -1), Wl, bl.reshape(1, -1), Wo, bo.reshape(1, 2))
